# Initial kernel scaffold; baseline (speedup 1.0000x reference)
import jax
import jax.numpy as jnp
from jax.experimental import pallas as pl


def kernel(atom_features, f_ini_atoms_bonds, atom_to_incoming_bonds, mapping, global_features, W_i, W_h, W_o, b_o, W_ffn1, b_ffn1, W_ffn2, b_ffn2, W_ffn3, b_ffn3):
    raise NotImplementedError("write your pallas kernel here")


# speedup vs baseline: 1.5412x; 1.5412x over previous
import functools

import jax
import jax.numpy as jnp
from jax import lax
from jax.experimental import pallas as pl
from jax.experimental.pallas import tpu as pltpu
from jax.experimental.pallas import tpu_sc as plsc

DEPTH = 5
NC, NS = 2, 16
NW = NC * NS
MAX_IN = 4


def _gather_sum_sc(table, idx_flat, m_rows, chunk):
    n_rows, hid = table.shape
    total_chunks = m_rows // chunk
    assert m_rows % chunk == 0
    assert (MAX_IN * chunk) % 8 == 0 and MAX_IN * chunk <= 128
    mesh = plsc.VectorSubcoreMesh(core_axis_name="c", subcore_axis_name="s",
                                  num_cores=NC, num_subcores=NS)

    @functools.partial(
        pl.kernel,
        out_type=jax.ShapeDtypeStruct((m_rows, hid), jnp.float32),
        mesh=mesh,
        scratch_types=[
            pltpu.VMEM((MAX_IN * chunk,), jnp.int32),
            pltpu.VMEM((MAX_IN * chunk, hid), jnp.float32),
            pltpu.VMEM((chunk, hid), jnp.float32),
            pltpu.SemaphoreType.DMA,
        ],
    )
    def gather_kernel(table_hbm, idx_hbm, out_hbm, idx_v, rows_v, acc_v, sem):
        wid = lax.axis_index("s") * NC + lax.axis_index("c")

        def chunk_body(ci, carry):
            base = ci * chunk
            pltpu.sync_copy(idx_hbm.at[pl.ds(MAX_IN * base, MAX_IN * chunk)],
                            idx_v)
            pltpu.async_copy(table_hbm.at[idx_v], rows_v, sem).wait()

            def row_body(r, c2):
                for s in range(hid // 16):
                    sl = pl.ds(s * 16, 16)
                    acc_v[r, sl] = (
                        (rows_v[MAX_IN * r, sl] + rows_v[MAX_IN * r + 1, sl])
                        + (rows_v[MAX_IN * r + 2, sl]
                           + rows_v[MAX_IN * r + 3, sl]))
                return c2

            lax.fori_loop(0, chunk, row_body, 0)
            pltpu.sync_copy(acc_v, out_hbm.at[pl.ds(base, chunk)])
            return carry

        n_mine = (total_chunks - wid + NW - 1) // NW

        def outer(k, carry):
            return chunk_body(wid + k * NW, carry)

        lax.fori_loop(0, n_mine, outer, 0)

    return gather_kernel(table, idx_flat)


def _proj_body(x_ref, w_ref, inp_ref, msg_ref):
    acc = jnp.dot(x_ref[...], w_ref[...], preferred_element_type=jnp.float32)
    inp_ref[...] = acc
    msg_ref[...] = jnp.maximum(acc, 0.0)


def _step_body(g_ref, inp_ref, w_ref, h_ref, msg_ref):
    h = inp_ref[...] + jnp.dot(g_ref[...], w_ref[...],
                               preferred_element_type=jnp.float32)
    h_ref[...] = h
    msg_ref[...] = jnp.maximum(h, 0.0)


def _tail_body(apm, af_ref, msgs_ref, gf_ref, woa_ref, wom_ref, bo_ref,
               w1g_ref, w1m_ref, b1_ref, w2_ref, b2_ref, w3t_ref, out_ref):
    hidden = jnp.maximum(
        jnp.dot(af_ref[...], woa_ref[...], preferred_element_type=jnp.float32)
        + jnp.dot(msgs_ref[...], wom_ref[...],
                  preferred_element_type=jnp.float32)
        + bo_ref[...], 0.0)
    n_atoms_blk, hid = hidden.shape
    mols = n_atoms_blk // apm
    mol = jnp.mean(hidden.reshape(mols, apm, hid), axis=1)
    h1 = jnp.maximum(
        jnp.dot(mol, w1m_ref[...], preferred_element_type=jnp.float32)
        + jnp.dot(gf_ref[...], w1g_ref[...],
                  preferred_element_type=jnp.float32)
        + b1_ref[...], 0.0)
    h2 = jnp.maximum(
        jnp.dot(h1, w2_ref[...], preferred_element_type=jnp.float32)
        + b2_ref[...], 0.0)
    out_ref[...] = jnp.sum(h2 * w3t_ref[...], axis=1, keepdims=True)


def kernel(atom_features, f_ini_atoms_bonds, atom_to_incoming_bonds, mapping,
           global_features, W_i, W_h, W_o, b_o, W_ffn1, b_ffn1, W_ffn2,
           b_ffn2, W_ffn3, b_ffn3):
    n_atoms, atom_f = atom_features.shape
    n_bonds, concat_f = f_ini_atoms_bonds.shape
    n_mols, gf_dim = global_features.shape
    hid = W_h.shape[0]
    apm = n_atoms // n_mols

    mb = 1000
    inp, message = pl.pallas_call(
        _proj_body,
        grid=(n_bonds // mb,),
        in_specs=[
            pl.BlockSpec((mb, concat_f), lambda i: (i, 0)),
            pl.BlockSpec((concat_f, hid), lambda i: (0, 0)),
        ],
        out_specs=[
            pl.BlockSpec((mb, hid), lambda i: (i, 0)),
            pl.BlockSpec((mb, hid), lambda i: (i, 0)),
        ],
        out_shape=[
            jax.ShapeDtypeStruct((n_bonds, hid), jnp.float32),
            jax.ShapeDtypeStruct((n_bonds, hid), jnp.float32),
        ],
    )(f_ini_atoms_bonds, W_i)

    map_flat = mapping.reshape(-1).astype(jnp.int32)
    step_call = pl.pallas_call(
        _step_body,
        grid=(n_bonds // mb,),
        in_specs=[
            pl.BlockSpec((mb, hid), lambda i: (i, 0)),
            pl.BlockSpec((mb, hid), lambda i: (i, 0)),
            pl.BlockSpec((hid, hid), lambda i: (0, 0)),
        ],
        out_specs=[
            pl.BlockSpec((mb, hid), lambda i: (i, 0)),
            pl.BlockSpec((mb, hid), lambda i: (i, 0)),
        ],
        out_shape=[
            jax.ShapeDtypeStruct((n_bonds, hid), jnp.float32),
            jax.ShapeDtypeStruct((n_bonds, hid), jnp.float32),
        ],
    )
    h_message = message
    for _ in range(1, DEPTH):
        gathered = _gather_sum_sc(message, map_flat, n_bonds, 16)
        h_message, message = step_call(gathered, inp, W_h)

    atib_flat = atom_to_incoming_bonds.reshape(-1).astype(jnp.int32)
    msgs_to_atoms = _gather_sum_sc(h_message, atib_flat, n_atoms, 16)

    mol_blk = 40
    atoms_blk = mol_blk * apm
    out = pl.pallas_call(
        functools.partial(_tail_body, apm),
        grid=(n_mols // mol_blk,),
        in_specs=[
            pl.BlockSpec((atoms_blk, atom_f), lambda i: (i, 0)),
            pl.BlockSpec((atoms_blk, hid), lambda i: (i, 0)),
            pl.BlockSpec((mol_blk, gf_dim), lambda i: (i, 0)),
            pl.BlockSpec((atom_f, hid), lambda i: (0, 0)),
            pl.BlockSpec((hid, hid), lambda i: (0, 0)),
            pl.BlockSpec((1, hid), lambda i: (0, 0)),
            pl.BlockSpec((gf_dim, hid), lambda i: (0, 0)),
            pl.BlockSpec((hid, hid), lambda i: (0, 0)),
            pl.BlockSpec((1, hid), lambda i: (0, 0)),
            pl.BlockSpec((hid, hid), lambda i: (0, 0)),
            pl.BlockSpec((1, hid), lambda i: (0, 0)),
            pl.BlockSpec((1, hid), lambda i: (0, 0)),
        ],
        out_specs=pl.BlockSpec((mol_blk, 1), lambda i: (i, 0)),
        out_shape=jax.ShapeDtypeStruct((n_mols, 1), jnp.float32),
    )(atom_features, msgs_to_atoms, global_features,
      W_o[:atom_f], W_o[atom_f:], b_o.reshape(1, hid),
      W_ffn1[hid:], W_ffn1[:hid], b_ffn1.reshape(1, hid),
      W_ffn2, b_ffn2.reshape(1, hid), W_ffn3.reshape(1, hid))
    return out + b_ffn3
